# Initial kernel scaffold; baseline (speedup 1.0000x reference)
import jax
import jax.numpy as jnp
from jax.experimental import pallas as pl


def kernel(queries, memory_bank):
    raise NotImplementedError("write your pallas kernel here")


# speedup vs baseline: 6.4208x; 6.4208x over previous
import jax
import jax.numpy as jnp
from jax.experimental import pallas as pl
from jax.experimental.pallas import tpu as pltpu

_B = 4
_P = 784
_D = 1024
_K = 8192
_BK = 512
_LG = 128


def _nn_kernel(q_ref, m_ref, dist_ref, img_ref, acc_ref):
    k = pl.program_id(1)
    nk = pl.num_programs(1)
    q = q_ref[...]
    m = m_ref[...]
    ones = jnp.ones((1, _D), jnp.float32)
    prod = jax.lax.dot_general(
        q, m, (((1,), (1,)), ((), ())), preferred_element_type=jnp.float32)
    m_sq = jax.lax.dot_general(
        ones, m * m, (((1,), (1,)), ((), ())),
        preferred_element_type=jnp.float32)
    d2 = m_sq - 2.0 * prod
    t = d2[:, 0:_LG]
    for j in range(1, _BK // _LG):
        t = jnp.minimum(t, d2[:, j * _LG:(j + 1) * _LG])

    @pl.when(k == 0)
    def _():
        acc_ref[...] = t

    @pl.when(k > 0)
    def _():
        acc_ref[...] = jnp.minimum(acc_ref[...], t)

    @pl.when(k == nk - 1)
    def _():
        q_sq = jax.lax.dot_general(
            q * q, ones, (((1,), (1,)), ((), ())),
            preferred_element_type=jnp.float32)
        mind2 = jnp.min(acc_ref[...], axis=1, keepdims=True)
        d2f = jnp.maximum(mind2 + q_sq, 0.0)
        dist = jnp.sqrt(jnp.maximum(d2f, 1e-12))
        dist_ref[...] = dist
        img_ref[...] = jnp.max(dist).reshape(1, 1, 1)


def _nn_call(queries, memory_bank, interpret=False):
    return pl.pallas_call(
        _nn_kernel,
        grid=(_B, _K // _BK),
        in_specs=[
            pl.BlockSpec((_P, _D), lambda i, k: (i, 0)),
            pl.BlockSpec((_BK, _D), lambda i, k: (k, 0)),
        ],
        out_specs=[
            pl.BlockSpec((_P, 1), lambda i, k: (i, 0)),
            pl.BlockSpec((1, 1, 1), lambda i, k: (i, 0, 0)),
        ],
        out_shape=[
            jax.ShapeDtypeStruct((_B * _P, 1), jnp.float32),
            jax.ShapeDtypeStruct((_B, 1, 1), jnp.float32),
        ],
        scratch_shapes=[pltpu.VMEM((_P, _LG), jnp.float32)],
        compiler_params=pltpu.CompilerParams(
            dimension_semantics=("parallel", "arbitrary")),
        interpret=interpret,
    )(queries, memory_bank)


def kernel(queries, memory_bank):
    dists, img = _nn_call(queries, memory_bank)
    patch_scores = dists.reshape(_B, _P)
    image_scores = img[:, 0, 0]
    return (patch_scores, image_scores)
